# baseline (speedup 1.0000x reference)
import functools

import jax
import jax.numpy as jnp
from jax import lax
from jax.experimental import pallas as pl
from jax.experimental.pallas import tpu as pltpu
from jax.experimental.pallas import tpu_sc as plsc

_J = 15
_K = 8
_D = 3


def _mask_body(f_ref, o_ref, g_ref, e_ref):
    jd = _J * _D
    n_dim, r_dim = g_ref.shape

    @pl.when(pl.program_id(0) == 0)
    def _init_expansion_consts():
        rr_n = lax.broadcasted_iota(jnp.int32, (n_dim, r_dim), 1) // jd
        nn = lax.broadcasted_iota(jnp.int32, (n_dim, r_dim), 0)
        g_ref[...] = (rr_n == nn).astype(jnp.float32)
        rr_j = lax.broadcasted_iota(jnp.int32, (_J, r_dim), 1) % jd // _D
        jj = lax.broadcasted_iota(jnp.int32, (_J, r_dim), 0)
        e_ref[...] = (rr_j == jj).astype(jnp.float32)

    f = f_ref[0]
    n = f.shape[-1]
    m = jnp.max(f, axis=(1, 2), keepdims=True)
    ex = jnp.exp(f - m)
    z = jnp.sum(ex, axis=(1, 2), keepdims=True)
    s = jnp.sum(ex * (f - m), axis=(1, 2), keepdims=True)
    ent = (jnp.log(z) - s / z)[:, 0, 0, :]

    j_sub = lax.broadcasted_iota(jnp.int32, (_J, n), 0)
    acc = jnp.zeros((_J, n), jnp.float32)
    for r in range(1, _J):
        other = jnp.concatenate([ent[r:], ent[:r]], axis=0)
        wrap = j_sub >= (_J - r)
        beats = (other > ent) | ((other == ent) & wrap)
        acc = acc + beats.astype(jnp.float32)
    mask = (acc < float(_K)).astype(jnp.float32)

    t1 = jnp.dot(mask, g_ref[...], preferred_element_type=jnp.float32)
    lane_mask = jnp.sum(t1 * e_ref[...], axis=0, keepdims=True)
    o_ref[0] = jnp.broadcast_to(lane_mask, o_ref.shape[1:])


def kernel(joint_features, data, k):
    del k
    B, N, J, T, Dt = joint_features.shape
    C = data.shape[-1]
    R = N * J * _D
    feats_t = jnp.transpose(joint_features, (0, 2, 4, 3, 1))
    data_t = jnp.transpose(data, (0, 2, 1))

    mask8 = pl.pallas_call(
        _mask_body,
        grid=(B,),
        in_specs=[pl.BlockSpec((1, J, _D, T, N), lambda i: (i, 0, 0, 0, 0))],
        out_specs=pl.BlockSpec((1, 8, R), lambda i: (i, 0, 0)),
        out_shape=jax.ShapeDtypeStruct((B, 8, R), jnp.float32),
        scratch_shapes=[
            pltpu.VMEM((N, R), jnp.float32),
            pltpu.VMEM((J, R), jnp.float32),
        ],
        compiler_params=pltpu.CompilerParams(
            dimension_semantics=("arbitrary",),
        ),
    )(feats_t)

    mesh = plsc.VectorSubcoreMesh(core_axis_name="c", subcore_axis_name="s")

    @functools.partial(
        pl.kernel,
        mesh=mesh,
        out_type=jax.ShapeDtypeStruct((B, C, R), jnp.float32),
        scratch_types=[
            pltpu.VMEM((8, R), jnp.float32),
            pltpu.VMEM((8, R), jnp.float32),
        ],
    )
    def _sc_masked_copy(data_hbm, mask_hbm, out_hbm, dbuf, mbuf):
        wid = lax.axis_index("s") * 2 + lax.axis_index("c")
        for bb in range(2):
            b = wid * 2 + bb
            pltpu.sync_copy(mask_hbm.at[b], mbuf)

            def chunk(ch, carry):
                pltpu.sync_copy(data_hbm.at[b, pl.ds(ch * 8, 8)], dbuf)

                def col(i, c2):
                    sl = pl.ds(i * 16, 16)
                    mv = mbuf[0, sl]
                    for c in range(8):
                        dbuf[c, sl] = dbuf[c, sl] * mv
                    return c2

                lax.fori_loop(0, R // 16, col, 0)
                pltpu.sync_copy(dbuf, out_hbm.at[b, pl.ds(ch * 8, 8)])
                return carry

            lax.fori_loop(0, C // 8, chunk, 0)

    out_t = _sc_masked_copy(data_t, mask8)
    return jnp.transpose(out_t, (0, 2, 1))


# speedup vs baseline: 2.3149x; 2.3149x over previous
import jax
import jax.numpy as jnp
from jax import lax
from jax.experimental import pallas as pl
from jax.experimental.pallas import tpu as pltpu

_J = 15
_K = 8
_D = 3


def _masked_select_body(f_ref, d_ref, o_ref, g_ref, e_ref):
    jd = _J * _D
    n_dim, r_dim = g_ref.shape

    @pl.when(pl.program_id(0) == 0)
    def _init_expansion_consts():
        rr_n = lax.broadcasted_iota(jnp.int32, (n_dim, r_dim), 1) // jd
        nn = lax.broadcasted_iota(jnp.int32, (n_dim, r_dim), 0)
        g_ref[...] = (rr_n == nn).astype(jnp.float32)
        rr_j = lax.broadcasted_iota(jnp.int32, (_J, r_dim), 1) % jd // _D
        jj = lax.broadcasted_iota(jnp.int32, (_J, r_dim), 0)
        e_ref[...] = (rr_j == jj).astype(jnp.float32)

    f = f_ref[0]
    n = f.shape[-1]
    m = jnp.max(f, axis=(1, 2), keepdims=True)
    ex = jnp.exp(f - m)
    z = jnp.sum(ex, axis=(1, 2), keepdims=True)
    s = jnp.sum(ex * (f - m), axis=(1, 2), keepdims=True)
    ent = (jnp.log(z) - s / z)[:, 0, 0, :]

    j_sub = lax.broadcasted_iota(jnp.int32, (_J, n), 0)
    acc = jnp.zeros((_J, n), jnp.float32)
    for r in range(1, _J):
        other = jnp.concatenate([ent[r:], ent[:r]], axis=0)
        wrap = j_sub >= (_J - r)
        beats = (other > ent) | ((other == ent) & wrap)
        acc = acc + beats.astype(jnp.float32)
    mask = (acc < float(_K)).astype(jnp.float32)

    t1 = jnp.dot(mask, g_ref[...], preferred_element_type=jnp.float32)
    lane_mask = jnp.sum(t1 * e_ref[...], axis=0, keepdims=True)
    o_ref[0] = d_ref[0] * lane_mask


def kernel(joint_features, data, k):
    del k
    B, N, J, T, Dt = joint_features.shape
    C = data.shape[-1]
    R = N * J * _D
    feats_t = jnp.transpose(joint_features, (0, 2, 4, 3, 1))
    data_t = jnp.transpose(data, (0, 2, 1))

    out_t = pl.pallas_call(
        _masked_select_body,
        grid=(B,),
        in_specs=[
            pl.BlockSpec((1, J, _D, T, N), lambda i: (i, 0, 0, 0, 0)),
            pl.BlockSpec((1, C, R), lambda i: (i, 0, 0)),
        ],
        out_specs=pl.BlockSpec((1, C, R), lambda i: (i, 0, 0)),
        out_shape=jax.ShapeDtypeStruct((B, C, R), data.dtype),
        scratch_shapes=[
            pltpu.VMEM((N, R), jnp.float32),
            pltpu.VMEM((J, R), jnp.float32),
        ],
        compiler_params=pltpu.CompilerParams(
            dimension_semantics=("arbitrary",),
        ),
    )(feats_t, data_t)

    return jnp.transpose(out_t, (0, 2, 1))
